# baseline (speedup 1.0000x reference)
import functools

import jax
import jax.numpy as jnp
from jax import lax
from jax.experimental import pallas as pl
from jax.experimental.pallas import tpu as pltpu
from jax.experimental.pallas import tpu_sc as plsc


@functools.cache
def _make_gather(V, D, B):
    info = plsc.get_sparse_core_info()
    NC, NS = info.num_cores, info.num_subcores
    NW = NC * NS
    assert B % (8 * NW) == 0
    b_per_w = B // NW
    mesh = plsc.VectorSubcoreMesh(core_axis_name="c", subcore_axis_name="s")

    @functools.partial(
        pl.kernel,
        mesh=mesh,
        out_type=jax.ShapeDtypeStruct((B, D), jnp.float32),
        scratch_types=[
            pltpu.VMEM((b_per_w,), jnp.int32),
            pltpu.VMEM((2, b_per_w // 2, D), jnp.float32),
            pltpu.SemaphoreType.DMA,
            pltpu.SemaphoreType.DMA,
            pltpu.SemaphoreType.DMA,
            pltpu.SemaphoreType.DMA,
        ],
    )
    def k(table_hbm, idx_hbm, out_hbm, idx_v, rows_v, g0, g1, s0, s1):
        wid = lax.axis_index("s") * NC + lax.axis_index("c")
        base = wid * b_per_w
        H = b_per_w // 2
        pltpu.sync_copy(idx_hbm.at[pl.ds(base, b_per_w)], idx_v)
        ga = pltpu.async_copy(table_hbm.at[idx_v.at[pl.ds(0, H)]], rows_v.at[0], g0)
        gb = pltpu.async_copy(table_hbm.at[idx_v.at[pl.ds(H, H)]], rows_v.at[1], g1)
        ga.wait()
        sa = pltpu.async_copy(rows_v.at[0], out_hbm.at[pl.ds(base, H)], s0)
        gb.wait()
        sb = pltpu.async_copy(rows_v.at[1], out_hbm.at[pl.ds(base + H, H)], s1)
        sa.wait()
        sb.wait()

    return k


@jax.jit
def kernel(inputs, table):
    idx = inputs.astype(jnp.int32)
    return _make_gather(table.shape[0], table.shape[1], idx.shape[0])(
        table, idx
    )


# speedup vs baseline: 2.0150x; 2.0150x over previous
import functools

import jax
import jax.numpy as jnp
from jax import lax
from jax.experimental import pallas as pl
from jax.experimental.pallas import tpu as pltpu

_BB = 1024


def _tc_body(idx_ref, table_ref, out_ref):
    idx = idx_ref[0, 0, :]
    V = table_ref.shape[0]
    onehot = (idx[:, None] == lax.broadcasted_iota(jnp.int32, (idx.shape[0], V), 1)).astype(jnp.float32)
    out_ref[...] = jax.lax.dot_general(
        onehot, table_ref[...],
        dimension_numbers=(((1,), (0,)), ((), ())),
        preferred_element_type=jnp.float32,
    )


@functools.cache
def _make_tc(V, D, B, BB):
    NB = B // BB

    def call(idx, table):
        idx3 = idx.reshape(NB, 1, BB)
        return pl.pallas_call(
            _tc_body,
            grid=(NB,),
            in_specs=[
                pl.BlockSpec((1, 1, BB), lambda i: (i, 0, 0)),
                pl.BlockSpec((V, D), lambda i: (0, 0)),
            ],
            out_specs=pl.BlockSpec((BB, D), lambda i: (i, 0)),
            out_shape=jax.ShapeDtypeStruct((B, D), jnp.float32),
        )(idx3, table)

    return call


@jax.jit
def kernel(inputs, table):
    idx = inputs.astype(jnp.int32)
    return _make_tc(table.shape[0], table.shape[1], idx.shape[0], _BB)(idx, table)


# speedup vs baseline: 2.3083x; 1.1456x over previous
import functools

import jax
import jax.numpy as jnp
from jax import lax
from jax.experimental import pallas as pl
from jax.experimental.pallas import tpu as pltpu

_BB = 2048


def _tc_body(idx_ref, table_ref, out_ref):
    idx = idx_ref[0, 0, :]
    V = table_ref.shape[0]
    onehot = (idx[:, None] == lax.broadcasted_iota(jnp.int32, (idx.shape[0], V), 1)).astype(jnp.float32)
    out_ref[...] = jax.lax.dot_general(
        onehot, table_ref[...],
        dimension_numbers=(((1,), (0,)), ((), ())),
        preferred_element_type=jnp.float32,
    )


@functools.cache
def _make_tc(V, D, B, BB):
    NB = B // BB

    def call(idx, table):
        idx3 = idx.reshape(NB, 1, BB)
        return pl.pallas_call(
            _tc_body,
            grid=(NB,),
            in_specs=[
                pl.BlockSpec((1, 1, BB), lambda i: (i, 0, 0)),
                pl.BlockSpec((V, D), lambda i: (0, 0)),
            ],
            out_specs=pl.BlockSpec((BB, D), lambda i: (i, 0)),
            out_shape=jax.ShapeDtypeStruct((B, D), jnp.float32),
        )(idx3, table)

    return call


@jax.jit
def kernel(inputs, table):
    idx = inputs.astype(jnp.int32)
    return _make_tc(table.shape[0], table.shape[1], idx.shape[0], _BB)(idx, table)


# speedup vs baseline: 2.4202x; 1.0484x over previous
import functools

import jax
import jax.numpy as jnp
from jax import lax
from jax.experimental import pallas as pl
from jax.experimental.pallas import tpu as pltpu

_BB = 4096


def _tc_body(idx_ref, table_ref, out_ref):
    idx = idx_ref[0, 0, :]
    V = table_ref.shape[0]
    onehot = (idx[:, None] == lax.broadcasted_iota(jnp.int32, (idx.shape[0], V), 1)).astype(jnp.float32)
    out_ref[...] = jax.lax.dot_general(
        onehot, table_ref[...],
        dimension_numbers=(((1,), (0,)), ((), ())),
        preferred_element_type=jnp.float32,
    )


@functools.cache
def _make_tc(V, D, B, BB):
    NB = B // BB

    def call(idx, table):
        idx3 = idx.reshape(NB, 1, BB)
        return pl.pallas_call(
            _tc_body,
            grid=(NB,),
            in_specs=[
                pl.BlockSpec((1, 1, BB), lambda i: (i, 0, 0)),
                pl.BlockSpec((V, D), lambda i: (0, 0)),
            ],
            out_specs=pl.BlockSpec((BB, D), lambda i: (i, 0)),
            out_shape=jax.ShapeDtypeStruct((B, D), jnp.float32),
        )(idx3, table)

    return call


@jax.jit
def kernel(inputs, table):
    idx = inputs.astype(jnp.int32)
    return _make_tc(table.shape[0], table.shape[1], idx.shape[0], _BB)(idx, table)
